# baseline (speedup 1.0000x reference)
import functools

import jax
import jax.numpy as jnp
import numpy as _np
from jax import lax
from jax.experimental import pallas as pl
from jax.experimental.pallas import tpu as pltpu
from jax.experimental.pallas import tpu_sc as plsc

N = 10000
R = 8
D_IN = 514
HID = 128
LBL = 16
E = 160000

NTILES = 32
B = 128
TPB = 40
EPT = B * TPB
EP = EPT * NTILES
HTOT = EPT * 16
NPAD = 640 * 16
RPT = 640
CH = 20

_perm = _np.zeros(HID, _np.int32)
for _c in range(HID // 32):
    for _k in range(16):
        _perm[32 * _c + 2 * _k] = 32 * _c + _k
        _perm[32 * _c + 2 * _k + 1] = 32 * _c + 16 + _k
_PERM = jnp.asarray(_perm)

_mesh = plsc.VectorSubcoreMesh(core_axis_name="c", subcore_axis_name="s")
_sc_params = pltpu.CompilerParams(needs_layout_passes=False,
                                  use_tc_tiling_on_sc=False)


@functools.partial(
    pl.kernel,
    out_type=jax.ShapeDtypeStruct((2, HTOT), jnp.float32),
    mesh=_mesh,
    compiler_params=_sc_params,
    scratch_types=[
        pltpu.VMEM((TPB, B), jnp.int32),
        pltpu.VMEM((B,), jnp.float32),
        pltpu.VMEM((EPT,), jnp.float32),
        pltpu.VMEM_SHARED((HTOT,), jnp.float32),
    ],
)
def _count_kernel(cidx_hbm, cnt2_hbm, cidxv, ones128, zbuf, hist):
    c = lax.axis_index("c")
    s = lax.axis_index("s")
    wid = c * 16 + s

    def fill(i, carry):
        zbuf[pl.ds(i * 16, 16)] = jnp.zeros((16,), jnp.float32)
        return carry

    lax.fori_loop(0, EPT // 16, fill, None)

    def fill1(i, carry):
        ones128[pl.ds(i * 16, 16)] = jnp.ones((16,), jnp.float32)
        return carry

    lax.fori_loop(0, B // 16, fill1, None)

    pltpu.sync_copy(zbuf, hist.at[pl.ds(s * EPT, EPT)])
    plsc.subcore_barrier()

    pltpu.sync_copy(cidx_hbm.at[pl.ds(wid * TPB, TPB)], cidxv)

    def body(b, carry):
        pltpu.sync_copy(ones128, hist.at[cidxv.at[b]], add=True)
        return carry

    lax.fori_loop(0, TPB, body, None)
    plsc.subcore_barrier()
    pltpu.sync_copy(hist.at[pl.ds(s * EPT, EPT)],
                    cnt2_hbm.at[c, pl.ds(s * EPT, EPT)])


@functools.partial(
    pl.kernel,
    out_type=[
        jax.ShapeDtypeStruct((EP // B, B), jnp.float32),
        jax.ShapeDtypeStruct((2, NPAD, HID), jnp.float32),
    ],
    mesh=_mesh,
    compiler_params=_sc_params,
    scratch_types=[
        pltpu.VMEM((CH, B), jnp.int32),
        pltpu.VMEM((CH, B), jnp.int32),
        pltpu.VMEM((CH, B), jnp.float32),
        pltpu.VMEM((B, HID // 2), jnp.float32),
        pltpu.VMEM((B, HID // 2), jnp.float32),
        pltpu.VMEM((B, HID), jnp.float32),
        pltpu.VMEM_SHARED((NPAD, HID), jnp.float32),
        pltpu.SemaphoreType.DMA,
        pltpu.SemaphoreType.DMA,
        pltpu.SemaphoreType.DMA,
    ],
)
def _edge1_kernel(gidx_hbm, dst_hbm, cidx_hbm, inv_hbm, h1_hbm,
                  w_hbm, acc_hbm,
                  gidxv, dstv, wbuf, bbuf0, bbuf1, fbuf, acc,
                  wsem, g0, g1):
    c = lax.axis_index("c")
    s = lax.axis_index("s")
    wid = c * 16 + s

    def zb(j, carry):
        for v in range(HID // 16):
            fbuf[j, pl.ds(v * 16, 16)] = jnp.zeros((16,), jnp.float32)
        return carry

    lax.fori_loop(0, B, zb, None)

    def za(k, carry):
        pltpu.sync_copy(fbuf, acc.at[pl.ds(s * RPT + k * B, B)])
        return carry

    lax.fori_loop(0, RPT // B, za, None)
    plsc.subcore_barrier()

    shift16 = jnp.full((16,), 16, jnp.uint32)
    himask = jnp.full((16,), 0xFFFF0000, jnp.uint32)

    def _scale(bbuf, b):
        def grp(g, carry):
            for l in range(4):
                j = g * 4 + l
                wj = plsc.load_gather(
                    wbuf, [jnp.full((16,), b, jnp.int32),
                           jnp.full((16,), j, jnp.int32)])
                for v in range(HID // 32):
                    u = plsc.bitcast(bbuf[j, pl.ds(v * 16, 16)], jnp.uint32)
                    lo = plsc.bitcast(lax.shift_left(u, shift16), jnp.float32)
                    hi = plsc.bitcast(jnp.bitwise_and(u, himask), jnp.float32)
                    fbuf[j, pl.ds(v * 32, 16)] = lo * wj
                    fbuf[j, pl.ds(v * 32 + 16, 16)] = hi * wj
            return carry

        lax.fori_loop(0, B // 4, grp, None)

    def chunk(k, carry):
        rb = wid * TPB + k * CH
        pltpu.sync_copy(cidx_hbm.at[pl.ds(rb, CH)], gidxv)

        def wfire(b, icarry):
            pltpu.async_copy(inv_hbm.at[gidxv.at[b]], wbuf.at[b], wsem)
            return icarry

        lax.fori_loop(0, CH, wfire, None)

        def wdrain(b, icarry):
            pltpu.make_async_copy(inv_hbm.at[gidxv.at[b]], wbuf.at[b],
                                  wsem).wait()
            return icarry

        lax.fori_loop(0, CH, wdrain, None)
        pltpu.sync_copy(wbuf, w_hbm.at[pl.ds(rb, CH)])

        pltpu.sync_copy(gidx_hbm.at[pl.ds(rb, CH)], gidxv)
        pltpu.sync_copy(dst_hbm.at[pl.ds(rb, CH)], dstv)

        pltpu.async_copy(h1_hbm.at[gidxv.at[0]], bbuf0, g0)
        pltpu.async_copy(h1_hbm.at[gidxv.at[1]], bbuf1, g1)

        def body(bb, icarry):
            b0 = 2 * bb
            b1 = 2 * bb + 1
            pltpu.make_async_copy(h1_hbm.at[gidxv.at[b0]], bbuf0, g0).wait()
            _scale(bbuf0, b0)
            pltpu.sync_copy(fbuf, acc.at[dstv.at[b0]], add=True)

            @pl.when(b0 + 2 < CH)
            def _():
                pltpu.async_copy(h1_hbm.at[gidxv.at[b0 + 2]], bbuf0, g0)

            pltpu.make_async_copy(h1_hbm.at[gidxv.at[b1]], bbuf1, g1).wait()
            _scale(bbuf1, b1)
            pltpu.sync_copy(fbuf, acc.at[dstv.at[b1]], add=True)

            @pl.when(b1 + 2 < CH)
            def _():
                pltpu.async_copy(h1_hbm.at[gidxv.at[b1 + 2]], bbuf1, g1)

            return icarry

        lax.fori_loop(0, CH // 2, body, None)
        return carry

    lax.fori_loop(0, TPB // CH, chunk, None)
    plsc.subcore_barrier()
    pltpu.sync_copy(acc.at[pl.ds(s * RPT, RPT)],
                    acc_hbm.at[c, pl.ds(s * RPT, RPT)])


@functools.partial(
    pl.kernel,
    out_type=jax.ShapeDtypeStruct((2, NPAD, LBL), jnp.float32),
    mesh=_mesh,
    compiler_params=_sc_params,
    scratch_types=[
        pltpu.VMEM((CH, B), jnp.int32),
        pltpu.VMEM((CH, B), jnp.int32),
        pltpu.VMEM((CH, B), jnp.float32),
        pltpu.VMEM((B, LBL), jnp.float32),
        pltpu.VMEM((B, LBL), jnp.float32),
        pltpu.VMEM_SHARED((NPAD, LBL), jnp.float32),
        pltpu.SemaphoreType.DMA,
        pltpu.SemaphoreType.DMA,
        pltpu.SemaphoreType.DMA,
        pltpu.SemaphoreType.DMA,
        pltpu.SemaphoreType.DMA,
    ],
)
def _edge2_kernel(gidx_hbm, dst_hbm, w_hbm, h2_hbm, acc_hbm,
                  gidxv, dstv, wbuf, buf0, buf1, acc,
                  wsem, g0, g1, s0, s1):
    c = lax.axis_index("c")
    s = lax.axis_index("s")
    wid = c * 16 + s

    def zb(j, carry):
        buf0[j, pl.ds(0, 16)] = jnp.zeros((16,), jnp.float32)
        return carry

    lax.fori_loop(0, B, zb, None)

    def za(k, carry):
        pltpu.sync_copy(buf0, acc.at[pl.ds(s * RPT + k * B, B)])
        return carry

    lax.fori_loop(0, RPT // B, za, None)
    plsc.subcore_barrier()

    def _scale(buf, b):
        def grp(g, carry):
            for l in range(8):
                j = g * 8 + l
                wj = plsc.load_gather(
                    wbuf, [jnp.full((16,), b, jnp.int32),
                           jnp.full((16,), j, jnp.int32)])
                buf[j, pl.ds(0, 16)] = buf[j, pl.ds(0, 16)] * wj
            return carry

        lax.fori_loop(0, B // 8, grp, None)

    def chunk(k, carry):
        rb = wid * TPB + k * CH
        pltpu.sync_copy(w_hbm.at[pl.ds(rb, CH)], wbuf)
        pltpu.sync_copy(gidx_hbm.at[pl.ds(rb, CH)], gidxv)
        pltpu.sync_copy(dst_hbm.at[pl.ds(rb, CH)], dstv)

        pltpu.async_copy(h2_hbm.at[gidxv.at[0]], buf0, g0)
        pltpu.async_copy(h2_hbm.at[gidxv.at[1]], buf1, g1)

        def body(bb, icarry):
            b0 = 2 * bb
            b1 = 2 * bb + 1
            pltpu.make_async_copy(h2_hbm.at[gidxv.at[b0]], buf0, g0).wait()
            _scale(buf0, b0)
            pltpu.async_copy(buf0, acc.at[dstv.at[b0]], s0, add=True)
            pltpu.make_async_copy(h2_hbm.at[gidxv.at[b1]], buf1, g1).wait()
            _scale(buf1, b1)
            pltpu.async_copy(buf1, acc.at[dstv.at[b1]], s1, add=True)
            pltpu.make_async_copy(buf0, acc.at[dstv.at[b0]], s0).wait()

            @pl.when(b0 + 2 < CH)
            def _():
                pltpu.async_copy(h2_hbm.at[gidxv.at[b0 + 2]], buf0, g0)

            pltpu.make_async_copy(buf1, acc.at[dstv.at[b1]], s1).wait()

            @pl.when(b1 + 2 < CH)
            def _():
                pltpu.async_copy(h2_hbm.at[gidxv.at[b1 + 2]], buf1, g1)

            return icarry

        lax.fori_loop(0, CH // 2, body, None)
        return carry

    lax.fori_loop(0, TPB // CH, chunk, None)
    plsc.subcore_barrier()
    pltpu.sync_copy(acc.at[pl.ds(s * RPT, RPT)],
                    acc_hbm.at[c, pl.ds(s * RPT, RPT)])


BN = 2000


def _inv_body(c_ref, o_ref):
    o_ref[...] = 1.0 / jnp.maximum(c_ref[0] + c_ref[1], 1.0)


def _mm1_body(x_ref, w_ref, o_ref):
    i = pl.program_id(0)
    xb = x_ref[pl.ds(i * BN, BN), :]
    y = jnp.dot(xb, w_ref[0], preferred_element_type=jnp.float32)
    lo = lax.bitcast_convert_type(y[:, :HID // 2].astype(jnp.bfloat16),
                                  jnp.uint16).astype(jnp.uint32)
    hi = lax.bitcast_convert_type(y[:, HID // 2:].astype(jnp.bfloat16),
                                  jnp.uint16).astype(jnp.uint32)
    packed = jnp.bitwise_or(jnp.left_shift(hi, 16), lo)
    o_ref[...] = lax.bitcast_convert_type(packed, jnp.float32)


def _base1_body(x_ref, w_ref, o_ref):
    o_ref[...] = jnp.dot(x_ref[...], w_ref[...],
                         preferred_element_type=jnp.float32)


def _l2_body(b1_ref, acc_ref, bias1_ref, w2_ref, h2_ref):
    out1 = b1_ref[...] + bias1_ref[0] + acc_ref[0] + acc_ref[1]
    out1 = jnp.maximum(out1, 0.0)
    h2_ref[...] = jnp.dot(out1, w2_ref[0],
                          preferred_element_type=jnp.float32)


def _out_body(h2_ref, acc_ref, bias2_ref, o_ref):
    z = h2_ref[...] + bias2_ref[0] + acc_ref[0] + acc_ref[1]
    o_ref[...] = jax.nn.sigmoid(z)


def kernel(edge_index, edge_type, emb, W1, root1, bias1, W2, root2, bias2):
    src = edge_index[0]
    dst = edge_index[1]
    gidx = edge_type * N + src
    cidx = edge_type * N + dst

    pad = EP - E
    gidx2d = jnp.concatenate(
        [gidx, jnp.zeros((pad,), jnp.int32)]).reshape(EP // B, B)
    cidx2d = jnp.concatenate(
        [cidx, jnp.full((pad,), R * N, jnp.int32)]).reshape(EP // B, B)
    dst2d = jnp.concatenate(
        [dst, jnp.full((pad,), N, jnp.int32)]).reshape(EP // B, B)

    W1s = jnp.concatenate(
        [W1[:, :, _PERM[0::2]], W1[:, :, _PERM[1::2]]],
        axis=2).astype(jnp.bfloat16)
    W2cat = jnp.concatenate([W2, root2[None]], axis=0)
    bias1b = jnp.broadcast_to(bias1, (8, HID))
    bias2b = jnp.broadcast_to(bias2, (8, LBL))
    emb16 = emb.astype(jnp.bfloat16)

    cnt2 = _count_kernel(cidx2d)

    inv = pl.pallas_call(
        _inv_body,
        out_shape=jax.ShapeDtypeStruct((HTOT // 128, 128), jnp.float32),
    )(cnt2.reshape(2, HTOT // 128, 128)).reshape(HTOT)

    h1t = pl.pallas_call(
        _mm1_body,
        grid=(N // BN, R),
        in_specs=[
            pl.BlockSpec((N, D_IN), lambda i, r: (0, 0)),
            pl.BlockSpec((1, D_IN, HID), lambda i, r: (r, 0, 0)),
        ],
        out_specs=pl.BlockSpec((BN, HID // 2),
                               lambda i, r: (r * (N // BN) + i, 0)),
        out_shape=jax.ShapeDtypeStruct((R * N, HID // 2), jnp.float32),
    )(emb16, W1s)

    base1 = pl.pallas_call(
        _base1_body,
        grid=(N // BN,),
        in_specs=[
            pl.BlockSpec((BN, D_IN), lambda i: (i, 0)),
            pl.BlockSpec((D_IN, HID), lambda i: (0, 0)),
        ],
        out_specs=pl.BlockSpec((BN, HID), lambda i: (i, 0)),
        out_shape=jax.ShapeDtypeStruct((N, HID), jnp.float32),
    )(emb16, root1.astype(jnp.bfloat16))

    w_e, acc1p = _edge1_kernel(gidx2d, dst2d, cidx2d, inv, h1t)

    h2t = pl.pallas_call(
        _l2_body,
        grid=(N // BN, R + 1),
        in_specs=[
            pl.BlockSpec((BN, HID), lambda i, r: (i, 0)),
            pl.BlockSpec((2, BN, HID), lambda i, r: (0, i, 0)),
            pl.BlockSpec((8, HID), lambda i, r: (0, 0)),
            pl.BlockSpec((1, HID, LBL), lambda i, r: (r, 0, 0)),
        ],
        out_specs=pl.BlockSpec((BN, LBL),
                               lambda i, r: (r * (N // BN) + i, 0)),
        out_shape=jax.ShapeDtypeStruct(((R + 1) * N, LBL), jnp.float32),
    )(base1, acc1p, bias1b, W2cat)

    acc2p = _edge2_kernel(gidx2d, dst2d, w_e, h2t)

    out = pl.pallas_call(
        _out_body,
        grid=(N // BN,),
        in_specs=[
            pl.BlockSpec((BN, LBL), lambda i: (R * (N // BN) + i, 0)),
            pl.BlockSpec((2, BN, LBL), lambda i: (0, i, 0)),
            pl.BlockSpec((8, LBL), lambda i: (0, 0)),
        ],
        out_specs=pl.BlockSpec((BN, LBL), lambda i: (i, 0)),
        out_shape=jax.ShapeDtypeStruct((N, LBL), jnp.float32),
    )(h2t, acc2p, bias2b)

    return out


# speedup vs baseline: 1.0624x; 1.0624x over previous
import functools

import jax
import jax.numpy as jnp
import numpy as _np
from jax import lax
from jax.experimental import pallas as pl
from jax.experimental.pallas import tpu as pltpu
from jax.experimental.pallas import tpu_sc as plsc

N = 10000
R = 8
D_IN = 514
HID = 128
LBL = 16
E = 160000

NTILES = 32
B = 128
TPB = 40
EPT = B * TPB
EP = EPT * NTILES
HTOT = EPT * 16
NPAD = 640 * 16
RPT = 640
CH = 20

_perm = _np.zeros(HID, _np.int32)
for _c in range(HID // 32):
    for _k in range(16):
        _perm[32 * _c + 2 * _k] = 32 * _c + _k
        _perm[32 * _c + 2 * _k + 1] = 32 * _c + 16 + _k
_PERM = jnp.asarray(_perm)

_mesh = plsc.VectorSubcoreMesh(core_axis_name="c", subcore_axis_name="s")
_sc_params = pltpu.CompilerParams(needs_layout_passes=False,
                                  use_tc_tiling_on_sc=False)


@functools.partial(
    pl.kernel,
    out_type=jax.ShapeDtypeStruct((2, HTOT), jnp.float32),
    mesh=_mesh,
    compiler_params=_sc_params,
    scratch_types=[
        pltpu.VMEM((TPB, B), jnp.int32),
        pltpu.VMEM((B,), jnp.float32),
        pltpu.VMEM((EPT,), jnp.float32),
        pltpu.VMEM_SHARED((HTOT,), jnp.float32),
    ],
)
def _count_kernel(cidx_hbm, cnt2_hbm, cidxv, ones128, zbuf, hist):
    c = lax.axis_index("c")
    s = lax.axis_index("s")
    wid = c * 16 + s

    def fill(i, carry):
        zbuf[pl.ds(i * 16, 16)] = jnp.zeros((16,), jnp.float32)
        return carry

    lax.fori_loop(0, EPT // 16, fill, None)

    def fill1(i, carry):
        ones128[pl.ds(i * 16, 16)] = jnp.ones((16,), jnp.float32)
        return carry

    lax.fori_loop(0, B // 16, fill1, None)

    pltpu.sync_copy(zbuf, hist.at[pl.ds(s * EPT, EPT)])
    plsc.subcore_barrier()

    pltpu.sync_copy(cidx_hbm.at[pl.ds(wid * TPB, TPB)], cidxv)

    def body(b, carry):
        pltpu.sync_copy(ones128, hist.at[cidxv.at[b]], add=True)
        return carry

    lax.fori_loop(0, TPB, body, None)
    plsc.subcore_barrier()
    pltpu.sync_copy(hist.at[pl.ds(s * EPT, EPT)],
                    cnt2_hbm.at[c, pl.ds(s * EPT, EPT)])


@functools.partial(
    pl.kernel,
    out_type=[
        jax.ShapeDtypeStruct((EP // B, B), jnp.float32),
        jax.ShapeDtypeStruct((2, NPAD, HID), jnp.float32),
    ],
    mesh=_mesh,
    compiler_params=_sc_params,
    scratch_types=[
        pltpu.VMEM((CH, B), jnp.int32),
        pltpu.VMEM((CH, B), jnp.int32),
        pltpu.VMEM((CH, B), jnp.float32),
        pltpu.VMEM((B, HID // 2), jnp.float32),
        pltpu.VMEM((B, HID // 2), jnp.float32),
        pltpu.VMEM((B, HID), jnp.float32),
        pltpu.VMEM_SHARED((NPAD, HID), jnp.float32),
        pltpu.SemaphoreType.DMA,
        pltpu.SemaphoreType.DMA,
        pltpu.SemaphoreType.DMA,
    ],
)
def _edge1_kernel(gidx_hbm, dst_hbm, cidx_hbm, inv_hbm, h1_hbm,
                  w_hbm, acc_hbm,
                  gidxv, dstv, wbuf, bbuf0, bbuf1, fbuf, acc,
                  wsem, g0, g1):
    c = lax.axis_index("c")
    s = lax.axis_index("s")
    wid = c * 16 + s

    def zb(j, carry):
        for v in range(HID // 16):
            fbuf[j, pl.ds(v * 16, 16)] = jnp.zeros((16,), jnp.float32)
        return carry

    lax.fori_loop(0, B, zb, None)

    def za(k, carry):
        pltpu.sync_copy(fbuf, acc.at[pl.ds(s * RPT + k * B, B)])
        return carry

    lax.fori_loop(0, RPT // B, za, None)
    plsc.subcore_barrier()

    shift16 = jnp.full((16,), 16, jnp.uint32)
    himask = jnp.full((16,), 0xFFFF0000, jnp.uint32)

    def _scale(bbuf, b):
        def grp(g, carry):
            for l in range(4):
                j = g * 4 + l
                wj = plsc.load_gather(
                    wbuf, [jnp.full((16,), b, jnp.int32),
                           jnp.full((16,), j, jnp.int32)])
                for v in range(HID // 32):
                    u = plsc.bitcast(bbuf[j, pl.ds(v * 16, 16)], jnp.uint32)
                    lo = plsc.bitcast(lax.shift_left(u, shift16), jnp.float32)
                    hi = plsc.bitcast(jnp.bitwise_and(u, himask), jnp.float32)
                    fbuf[j, pl.ds(v * 32, 16)] = lo * wj
                    fbuf[j, pl.ds(v * 32 + 16, 16)] = hi * wj
            return carry

        lax.fori_loop(0, B // 4, grp, None)

    def chunk(k, carry):
        rb = wid * TPB + k * CH
        pltpu.sync_copy(cidx_hbm.at[pl.ds(rb, CH)], gidxv)

        def wfire(b, icarry):
            pltpu.async_copy(inv_hbm.at[gidxv.at[b]], wbuf.at[b], wsem)
            return icarry

        lax.fori_loop(0, CH, wfire, None)

        def wdrain(b, icarry):
            pltpu.make_async_copy(inv_hbm.at[gidxv.at[b]], wbuf.at[b],
                                  wsem).wait()
            return icarry

        lax.fori_loop(0, CH, wdrain, None)
        pltpu.sync_copy(wbuf, w_hbm.at[pl.ds(rb, CH)])

        pltpu.sync_copy(gidx_hbm.at[pl.ds(rb, CH)], gidxv)
        pltpu.sync_copy(dst_hbm.at[pl.ds(rb, CH)], dstv)

        pltpu.async_copy(h1_hbm.at[gidxv.at[0]], bbuf0, g0)
        pltpu.async_copy(h1_hbm.at[gidxv.at[1]], bbuf1, g1)

        def body(bb, icarry):
            b0 = 2 * bb
            b1 = 2 * bb + 1
            pltpu.make_async_copy(h1_hbm.at[gidxv.at[b0]], bbuf0, g0).wait()
            _scale(bbuf0, b0)
            pltpu.sync_copy(fbuf, acc.at[dstv.at[b0]], add=True)

            @pl.when(b0 + 2 < CH)
            def _():
                pltpu.async_copy(h1_hbm.at[gidxv.at[b0 + 2]], bbuf0, g0)

            pltpu.make_async_copy(h1_hbm.at[gidxv.at[b1]], bbuf1, g1).wait()
            _scale(bbuf1, b1)
            pltpu.sync_copy(fbuf, acc.at[dstv.at[b1]], add=True)

            @pl.when(b1 + 2 < CH)
            def _():
                pltpu.async_copy(h1_hbm.at[gidxv.at[b1 + 2]], bbuf1, g1)

            return icarry

        lax.fori_loop(0, CH // 2, body, None)
        return carry

    lax.fori_loop(0, TPB // CH, chunk, None)
    plsc.subcore_barrier()
    pltpu.sync_copy(acc.at[pl.ds(s * RPT, RPT)],
                    acc_hbm.at[c, pl.ds(s * RPT, RPT)])


@functools.partial(
    pl.kernel,
    out_type=jax.ShapeDtypeStruct((2, NPAD, LBL), jnp.float32),
    mesh=_mesh,
    compiler_params=_sc_params,
    scratch_types=[
        pltpu.VMEM((CH, B), jnp.int32),
        pltpu.VMEM((CH, B), jnp.int32),
        pltpu.VMEM((CH, B), jnp.float32),
        pltpu.VMEM((B, LBL), jnp.float32),
        pltpu.VMEM((B, LBL), jnp.float32),
        pltpu.VMEM_SHARED((NPAD, LBL), jnp.float32),
        pltpu.SemaphoreType.DMA,
        pltpu.SemaphoreType.DMA,
        pltpu.SemaphoreType.DMA,
        pltpu.SemaphoreType.DMA,
        pltpu.SemaphoreType.DMA,
    ],
)
def _edge2_kernel(gidx_hbm, dst_hbm, w_hbm, h2_hbm, acc_hbm,
                  gidxv, dstv, wbuf, buf0, buf1, acc,
                  wsem, g0, g1, s0, s1):
    c = lax.axis_index("c")
    s = lax.axis_index("s")
    wid = c * 16 + s

    def zb(j, carry):
        buf0[j, pl.ds(0, 16)] = jnp.zeros((16,), jnp.float32)
        return carry

    lax.fori_loop(0, B, zb, None)

    def za(k, carry):
        pltpu.sync_copy(buf0, acc.at[pl.ds(s * RPT + k * B, B)])
        return carry

    lax.fori_loop(0, RPT // B, za, None)
    plsc.subcore_barrier()

    def _scale(buf, b):
        def grp(g, carry):
            for l in range(8):
                j = g * 8 + l
                wj = plsc.load_gather(
                    wbuf, [jnp.full((16,), b, jnp.int32),
                           jnp.full((16,), j, jnp.int32)])
                buf[j, pl.ds(0, 16)] = buf[j, pl.ds(0, 16)] * wj
            return carry

        lax.fori_loop(0, B // 8, grp, None)

    def chunk(k, carry):
        rb = wid * TPB + k * CH
        pltpu.sync_copy(w_hbm.at[pl.ds(rb, CH)], wbuf)
        pltpu.sync_copy(gidx_hbm.at[pl.ds(rb, CH)], gidxv)
        pltpu.sync_copy(dst_hbm.at[pl.ds(rb, CH)], dstv)

        pltpu.async_copy(h2_hbm.at[gidxv.at[0]], buf0, g0)
        pltpu.async_copy(h2_hbm.at[gidxv.at[1]], buf1, g1)

        def body(bb, icarry):
            b0 = 2 * bb
            b1 = 2 * bb + 1
            pltpu.make_async_copy(h2_hbm.at[gidxv.at[b0]], buf0, g0).wait()
            _scale(buf0, b0)
            pltpu.async_copy(buf0, acc.at[dstv.at[b0]], s0, add=True)
            pltpu.make_async_copy(h2_hbm.at[gidxv.at[b1]], buf1, g1).wait()
            _scale(buf1, b1)
            pltpu.async_copy(buf1, acc.at[dstv.at[b1]], s1, add=True)
            pltpu.make_async_copy(buf0, acc.at[dstv.at[b0]], s0).wait()

            @pl.when(b0 + 2 < CH)
            def _():
                pltpu.async_copy(h2_hbm.at[gidxv.at[b0 + 2]], buf0, g0)

            pltpu.make_async_copy(buf1, acc.at[dstv.at[b1]], s1).wait()

            @pl.when(b1 + 2 < CH)
            def _():
                pltpu.async_copy(h2_hbm.at[gidxv.at[b1 + 2]], buf1, g1)

            return icarry

        lax.fori_loop(0, CH // 2, body, None)
        return carry

    lax.fori_loop(0, TPB // CH, chunk, None)
    plsc.subcore_barrier()
    pltpu.sync_copy(acc.at[pl.ds(s * RPT, RPT)],
                    acc_hbm.at[c, pl.ds(s * RPT, RPT)])


BN = 2000


def _inv_body(c_ref, o_ref):
    o_ref[...] = 1.0 / jnp.maximum(c_ref[0] + c_ref[1], 1.0)


def _mm1_body(x_ref, w_ref, o_ref):
    i = pl.program_id(0)
    xb = x_ref[pl.ds(i * BN, BN), :]
    for r in range(R):
        y = jnp.dot(xb, w_ref[r], preferred_element_type=jnp.float32)
        lo = lax.bitcast_convert_type(y[:, :HID // 2].astype(jnp.bfloat16),
                                      jnp.uint16).astype(jnp.uint32)
        hi = lax.bitcast_convert_type(y[:, HID // 2:].astype(jnp.bfloat16),
                                      jnp.uint16).astype(jnp.uint32)
        packed = jnp.bitwise_or(jnp.left_shift(hi, 16), lo)
        o_ref[r] = lax.bitcast_convert_type(packed, jnp.float32)


def _base1_body(x_ref, w_ref, o_ref):
    o_ref[...] = jnp.dot(x_ref[...], w_ref[...],
                         preferred_element_type=jnp.float32)


def _l2_body(b1_ref, acc_ref, bias1_ref, w2_ref, h2_ref):
    out1 = b1_ref[...] + bias1_ref[0] + acc_ref[0] + acc_ref[1]
    out1 = jnp.maximum(out1, 0.0)
    for r in range(R + 1):
        h2_ref[r] = jnp.dot(out1, w2_ref[r],
                            preferred_element_type=jnp.float32)


def _out_body(h2_ref, acc_ref, bias2_ref, o_ref):
    z = h2_ref[0] + bias2_ref[0] + acc_ref[0] + acc_ref[1]
    o_ref[...] = jax.nn.sigmoid(z)


def kernel(edge_index, edge_type, emb, W1, root1, bias1, W2, root2, bias2):
    src = edge_index[0]
    dst = edge_index[1]
    gidx = edge_type * N + src
    cidx = edge_type * N + dst

    pad = EP - E
    gidx2d = jnp.concatenate(
        [gidx, jnp.zeros((pad,), jnp.int32)]).reshape(EP // B, B)
    cidx2d = jnp.concatenate(
        [cidx, jnp.full((pad,), R * N, jnp.int32)]).reshape(EP // B, B)
    dst2d = jnp.concatenate(
        [dst, jnp.full((pad,), N, jnp.int32)]).reshape(EP // B, B)

    W1s = jnp.concatenate(
        [W1[:, :, _PERM[0::2]], W1[:, :, _PERM[1::2]]],
        axis=2).astype(jnp.bfloat16)
    W2cat = jnp.concatenate([W2, root2[None]], axis=0)
    bias1b = jnp.broadcast_to(bias1, (8, HID))
    bias2b = jnp.broadcast_to(bias2, (8, LBL))
    emb16 = emb.astype(jnp.bfloat16)

    cnt2 = _count_kernel(cidx2d)

    inv = pl.pallas_call(
        _inv_body,
        out_shape=jax.ShapeDtypeStruct((HTOT // 128, 128), jnp.float32),
    )(cnt2.reshape(2, HTOT // 128, 128)).reshape(HTOT)

    h1b = pl.pallas_call(
        _mm1_body,
        grid=(N // BN,),
        in_specs=[
            pl.BlockSpec((N, D_IN), lambda i: (0, 0)),
            pl.BlockSpec((R, D_IN, HID), lambda i: (0, 0, 0)),
        ],
        out_specs=pl.BlockSpec((R, BN, HID // 2), lambda i: (0, i, 0)),
        out_shape=jax.ShapeDtypeStruct((R, N, HID // 2), jnp.float32),
    )(emb16, W1s)

    base1 = pl.pallas_call(
        _base1_body,
        grid=(N // BN,),
        in_specs=[
            pl.BlockSpec((BN, D_IN), lambda i: (i, 0)),
            pl.BlockSpec((D_IN, HID), lambda i: (0, 0)),
        ],
        out_specs=pl.BlockSpec((BN, HID), lambda i: (i, 0)),
        out_shape=jax.ShapeDtypeStruct((N, HID), jnp.float32),
    )(emb16, root1.astype(jnp.bfloat16))

    w_e, acc1p = _edge1_kernel(gidx2d, dst2d, cidx2d, inv,
                               h1b.reshape(R * N, HID // 2))

    h2b = pl.pallas_call(
        _l2_body,
        grid=(N // BN,),
        in_specs=[
            pl.BlockSpec((BN, HID), lambda i: (i, 0)),
            pl.BlockSpec((2, BN, HID), lambda i: (0, i, 0)),
            pl.BlockSpec((8, HID), lambda i: (0, 0)),
            pl.BlockSpec((R + 1, HID, LBL), lambda i: (0, 0, 0)),
        ],
        out_specs=pl.BlockSpec((R + 1, BN, LBL), lambda i: (0, i, 0)),
        out_shape=jax.ShapeDtypeStruct((R + 1, N, LBL), jnp.float32),
    )(base1, acc1p, bias1b, W2cat)

    acc2p = _edge2_kernel(gidx2d, dst2d, w_e,
                          h2b.reshape((R + 1) * N, LBL))

    out = pl.pallas_call(
        _out_body,
        grid=(N // BN,),
        in_specs=[
            pl.BlockSpec((1, BN, LBL), lambda i: (R, i, 0)),
            pl.BlockSpec((2, BN, LBL), lambda i: (0, i, 0)),
            pl.BlockSpec((8, LBL), lambda i: (0, 0)),
        ],
        out_specs=pl.BlockSpec((BN, LBL), lambda i: (i, 0)),
        out_shape=jax.ShapeDtypeStruct((N, LBL), jnp.float32),
    )(h2b, acc2p, bias2b)

    return out


# speedup vs baseline: 1.0836x; 1.0199x over previous
import functools

import jax
import jax.numpy as jnp
import numpy as _np
from jax import lax
from jax.experimental import pallas as pl
from jax.experimental.pallas import tpu as pltpu
from jax.experimental.pallas import tpu_sc as plsc

N = 10000
R = 8
D_IN = 514
HID = 128
LBL = 16
E = 160000

NTILES = 32
B = 128
TPB = 40
EPT = B * TPB
EP = EPT * NTILES
HTOT = EPT * 16
NPAD = 640 * 16
RPT = 640
CH = 40

_perm = _np.zeros(HID, _np.int32)
for _c in range(HID // 32):
    for _k in range(16):
        _perm[32 * _c + 2 * _k] = 32 * _c + _k
        _perm[32 * _c + 2 * _k + 1] = 32 * _c + 16 + _k
_PERM = jnp.asarray(_perm)

_mesh = plsc.VectorSubcoreMesh(core_axis_name="c", subcore_axis_name="s")
_sc_params = pltpu.CompilerParams(needs_layout_passes=False,
                                  use_tc_tiling_on_sc=False)


@functools.partial(
    pl.kernel,
    out_type=jax.ShapeDtypeStruct((2, HTOT), jnp.float32),
    mesh=_mesh,
    compiler_params=_sc_params,
    scratch_types=[
        pltpu.VMEM((TPB, B), jnp.int32),
        pltpu.VMEM((B,), jnp.float32),
        pltpu.VMEM((EPT,), jnp.float32),
        pltpu.VMEM_SHARED((HTOT,), jnp.float32),
    ],
)
def _count_kernel(cidx_hbm, cnt2_hbm, cidxv, ones128, zbuf, hist):
    c = lax.axis_index("c")
    s = lax.axis_index("s")
    wid = c * 16 + s

    def fill(i, carry):
        zbuf[pl.ds(i * 16, 16)] = jnp.zeros((16,), jnp.float32)
        return carry

    lax.fori_loop(0, EPT // 16, fill, None)

    def fill1(i, carry):
        ones128[pl.ds(i * 16, 16)] = jnp.ones((16,), jnp.float32)
        return carry

    lax.fori_loop(0, B // 16, fill1, None)

    pltpu.sync_copy(zbuf, hist.at[pl.ds(s * EPT, EPT)])
    plsc.subcore_barrier()

    pltpu.sync_copy(cidx_hbm.at[pl.ds(wid * TPB, TPB)], cidxv)

    def body(b, carry):
        pltpu.sync_copy(ones128, hist.at[cidxv.at[b]], add=True)
        return carry

    lax.fori_loop(0, TPB, body, None)
    plsc.subcore_barrier()
    pltpu.sync_copy(hist.at[pl.ds(s * EPT, EPT)],
                    cnt2_hbm.at[c, pl.ds(s * EPT, EPT)])


@functools.partial(
    pl.kernel,
    out_type=[
        jax.ShapeDtypeStruct((EP // B, B), jnp.float32),
        jax.ShapeDtypeStruct((2, NPAD, HID), jnp.float32),
    ],
    mesh=_mesh,
    compiler_params=_sc_params,
    scratch_types=[
        pltpu.VMEM((CH, B), jnp.int32),
        pltpu.VMEM((CH, B), jnp.int32),
        pltpu.VMEM((CH, B), jnp.float32),
        pltpu.VMEM((B, HID // 2), jnp.float32),
        pltpu.VMEM((B, HID // 2), jnp.float32),
        pltpu.VMEM((B, HID), jnp.float32),
        pltpu.VMEM_SHARED((NPAD, HID), jnp.float32),
        pltpu.SemaphoreType.DMA,
        pltpu.SemaphoreType.DMA,
        pltpu.SemaphoreType.DMA,
    ],
)
def _edge1_kernel(gidx_hbm, dst_hbm, cidx_hbm, inv_hbm, h1_hbm,
                  w_hbm, acc_hbm,
                  gidxv, dstv, wbuf, bbuf0, bbuf1, fbuf, acc,
                  wsem, g0, g1):
    c = lax.axis_index("c")
    s = lax.axis_index("s")
    wid = c * 16 + s

    def zb(j, carry):
        for v in range(HID // 16):
            fbuf[j, pl.ds(v * 16, 16)] = jnp.zeros((16,), jnp.float32)
        return carry

    lax.fori_loop(0, B, zb, None)

    def za(k, carry):
        pltpu.sync_copy(fbuf, acc.at[pl.ds(s * RPT + k * B, B)])
        return carry

    lax.fori_loop(0, RPT // B, za, None)
    plsc.subcore_barrier()

    shift16 = jnp.full((16,), 16, jnp.uint32)
    himask = jnp.full((16,), 0xFFFF0000, jnp.uint32)

    def _scale(bbuf, b):
        def grp(g, carry):
            for l in range(4):
                j = g * 4 + l
                wj = plsc.load_gather(
                    wbuf, [jnp.full((16,), b, jnp.int32),
                           jnp.full((16,), j, jnp.int32)])
                for v in range(HID // 32):
                    u = plsc.bitcast(bbuf[j, pl.ds(v * 16, 16)], jnp.uint32)
                    lo = plsc.bitcast(lax.shift_left(u, shift16), jnp.float32)
                    hi = plsc.bitcast(jnp.bitwise_and(u, himask), jnp.float32)
                    fbuf[j, pl.ds(v * 32, 16)] = lo * wj
                    fbuf[j, pl.ds(v * 32 + 16, 16)] = hi * wj
            return carry

        lax.fori_loop(0, B // 4, grp, None)

    def chunk(k, carry):
        rb = wid * TPB + k * CH
        pltpu.sync_copy(cidx_hbm.at[pl.ds(rb, CH)], gidxv)

        def wfire(b, icarry):
            pltpu.async_copy(inv_hbm.at[gidxv.at[b]], wbuf.at[b], wsem)
            return icarry

        lax.fori_loop(0, CH, wfire, None)

        def wdrain(b, icarry):
            pltpu.make_async_copy(inv_hbm.at[gidxv.at[b]], wbuf.at[b],
                                  wsem).wait()
            return icarry

        lax.fori_loop(0, CH, wdrain, None)
        pltpu.sync_copy(wbuf, w_hbm.at[pl.ds(rb, CH)])

        pltpu.sync_copy(gidx_hbm.at[pl.ds(rb, CH)], gidxv)
        pltpu.sync_copy(dst_hbm.at[pl.ds(rb, CH)], dstv)

        pltpu.async_copy(h1_hbm.at[gidxv.at[0]], bbuf0, g0)
        pltpu.async_copy(h1_hbm.at[gidxv.at[1]], bbuf1, g1)

        def body(bb, icarry):
            b0 = 2 * bb
            b1 = 2 * bb + 1
            pltpu.make_async_copy(h1_hbm.at[gidxv.at[b0]], bbuf0, g0).wait()
            _scale(bbuf0, b0)
            pltpu.sync_copy(fbuf, acc.at[dstv.at[b0]], add=True)

            @pl.when(b0 + 2 < CH)
            def _():
                pltpu.async_copy(h1_hbm.at[gidxv.at[b0 + 2]], bbuf0, g0)

            pltpu.make_async_copy(h1_hbm.at[gidxv.at[b1]], bbuf1, g1).wait()
            _scale(bbuf1, b1)
            pltpu.sync_copy(fbuf, acc.at[dstv.at[b1]], add=True)

            @pl.when(b1 + 2 < CH)
            def _():
                pltpu.async_copy(h1_hbm.at[gidxv.at[b1 + 2]], bbuf1, g1)

            return icarry

        lax.fori_loop(0, CH // 2, body, None)
        return carry

    lax.fori_loop(0, TPB // CH, chunk, None)
    plsc.subcore_barrier()
    pltpu.sync_copy(acc.at[pl.ds(s * RPT, RPT)],
                    acc_hbm.at[c, pl.ds(s * RPT, RPT)])


@functools.partial(
    pl.kernel,
    out_type=jax.ShapeDtypeStruct((2, NPAD, LBL), jnp.float32),
    mesh=_mesh,
    compiler_params=_sc_params,
    scratch_types=[
        pltpu.VMEM((CH, B), jnp.int32),
        pltpu.VMEM((CH, B), jnp.int32),
        pltpu.VMEM((CH, B), jnp.float32),
        pltpu.VMEM((B, LBL), jnp.float32),
        pltpu.VMEM((B, LBL), jnp.float32),
        pltpu.VMEM_SHARED((NPAD, LBL), jnp.float32),
        pltpu.SemaphoreType.DMA,
        pltpu.SemaphoreType.DMA,
        pltpu.SemaphoreType.DMA,
        pltpu.SemaphoreType.DMA,
        pltpu.SemaphoreType.DMA,
    ],
)
def _edge2_kernel(gidx_hbm, dst_hbm, w_hbm, h2_hbm, acc_hbm,
                  gidxv, dstv, wbuf, buf0, buf1, acc,
                  wsem, g0, g1, s0, s1):
    c = lax.axis_index("c")
    s = lax.axis_index("s")
    wid = c * 16 + s

    def zb(j, carry):
        buf0[j, pl.ds(0, 16)] = jnp.zeros((16,), jnp.float32)
        return carry

    lax.fori_loop(0, B, zb, None)

    def za(k, carry):
        pltpu.sync_copy(buf0, acc.at[pl.ds(s * RPT + k * B, B)])
        return carry

    lax.fori_loop(0, RPT // B, za, None)
    plsc.subcore_barrier()

    def _scale(buf, b):
        def grp(g, carry):
            for l in range(8):
                j = g * 8 + l
                wj = plsc.load_gather(
                    wbuf, [jnp.full((16,), b, jnp.int32),
                           jnp.full((16,), j, jnp.int32)])
                buf[j, pl.ds(0, 16)] = buf[j, pl.ds(0, 16)] * wj
            return carry

        lax.fori_loop(0, B // 8, grp, None)

    def chunk(k, carry):
        rb = wid * TPB + k * CH
        pltpu.sync_copy(w_hbm.at[pl.ds(rb, CH)], wbuf)
        pltpu.sync_copy(gidx_hbm.at[pl.ds(rb, CH)], gidxv)
        pltpu.sync_copy(dst_hbm.at[pl.ds(rb, CH)], dstv)

        pltpu.async_copy(h2_hbm.at[gidxv.at[0]], buf0, g0)
        pltpu.async_copy(h2_hbm.at[gidxv.at[1]], buf1, g1)

        def body(bb, icarry):
            b0 = 2 * bb
            b1 = 2 * bb + 1
            pltpu.make_async_copy(h2_hbm.at[gidxv.at[b0]], buf0, g0).wait()
            _scale(buf0, b0)
            pltpu.async_copy(buf0, acc.at[dstv.at[b0]], s0, add=True)
            pltpu.make_async_copy(h2_hbm.at[gidxv.at[b1]], buf1, g1).wait()
            _scale(buf1, b1)
            pltpu.async_copy(buf1, acc.at[dstv.at[b1]], s1, add=True)
            pltpu.make_async_copy(buf0, acc.at[dstv.at[b0]], s0).wait()

            @pl.when(b0 + 2 < CH)
            def _():
                pltpu.async_copy(h2_hbm.at[gidxv.at[b0 + 2]], buf0, g0)

            pltpu.make_async_copy(buf1, acc.at[dstv.at[b1]], s1).wait()

            @pl.when(b1 + 2 < CH)
            def _():
                pltpu.async_copy(h2_hbm.at[gidxv.at[b1 + 2]], buf1, g1)

            return icarry

        lax.fori_loop(0, CH // 2, body, None)
        return carry

    lax.fori_loop(0, TPB // CH, chunk, None)
    plsc.subcore_barrier()
    pltpu.sync_copy(acc.at[pl.ds(s * RPT, RPT)],
                    acc_hbm.at[c, pl.ds(s * RPT, RPT)])


BN = 2000


def _inv_body(c_ref, o_ref):
    o_ref[...] = 1.0 / jnp.maximum(c_ref[0] + c_ref[1], 1.0)


def _mm1_body(x_ref, w_ref, o_ref):
    i = pl.program_id(0)
    xb = x_ref[pl.ds(i * BN, BN), :]
    for r in range(R):
        y = jnp.dot(xb, w_ref[r], preferred_element_type=jnp.float32)
        lo = lax.bitcast_convert_type(y[:, :HID // 2].astype(jnp.bfloat16),
                                      jnp.uint16).astype(jnp.uint32)
        hi = lax.bitcast_convert_type(y[:, HID // 2:].astype(jnp.bfloat16),
                                      jnp.uint16).astype(jnp.uint32)
        packed = jnp.bitwise_or(jnp.left_shift(hi, 16), lo)
        o_ref[r] = lax.bitcast_convert_type(packed, jnp.float32)


def _base1_body(x_ref, w_ref, o_ref):
    o_ref[...] = jnp.dot(x_ref[...], w_ref[...],
                         preferred_element_type=jnp.float32)


def _l2_body(b1_ref, acc_ref, bias1_ref, w2_ref, h2_ref):
    out1 = b1_ref[...] + bias1_ref[0] + acc_ref[0] + acc_ref[1]
    out1 = jnp.maximum(out1, 0.0)
    for r in range(R + 1):
        h2_ref[r] = jnp.dot(out1, w2_ref[r],
                            preferred_element_type=jnp.float32)


def _out_body(h2_ref, acc_ref, bias2_ref, o_ref):
    z = h2_ref[0] + bias2_ref[0] + acc_ref[0] + acc_ref[1]
    o_ref[...] = jax.nn.sigmoid(z)


def kernel(edge_index, edge_type, emb, W1, root1, bias1, W2, root2, bias2):
    src = edge_index[0]
    dst = edge_index[1]
    gidx = edge_type * N + src
    cidx = edge_type * N + dst

    pad = EP - E
    gidx2d = jnp.concatenate(
        [gidx, jnp.zeros((pad,), jnp.int32)]).reshape(EP // B, B)
    cidx2d = jnp.concatenate(
        [cidx, jnp.full((pad,), R * N, jnp.int32)]).reshape(EP // B, B)
    dst2d = jnp.concatenate(
        [dst, jnp.full((pad,), N, jnp.int32)]).reshape(EP // B, B)

    W1s = jnp.concatenate(
        [W1[:, :, _PERM[0::2]], W1[:, :, _PERM[1::2]]],
        axis=2).astype(jnp.bfloat16)
    W2cat = jnp.concatenate([W2, root2[None]], axis=0)
    bias1b = jnp.broadcast_to(bias1, (8, HID))
    bias2b = jnp.broadcast_to(bias2, (8, LBL))
    emb16 = emb.astype(jnp.bfloat16)

    cnt2 = _count_kernel(cidx2d)

    inv = pl.pallas_call(
        _inv_body,
        out_shape=jax.ShapeDtypeStruct((HTOT // 128, 128), jnp.float32),
    )(cnt2.reshape(2, HTOT // 128, 128)).reshape(HTOT)

    h1b = pl.pallas_call(
        _mm1_body,
        grid=(N // BN,),
        in_specs=[
            pl.BlockSpec((N, D_IN), lambda i: (0, 0)),
            pl.BlockSpec((R, D_IN, HID), lambda i: (0, 0, 0)),
        ],
        out_specs=pl.BlockSpec((R, BN, HID // 2), lambda i: (0, i, 0)),
        out_shape=jax.ShapeDtypeStruct((R, N, HID // 2), jnp.float32),
    )(emb16, W1s)

    base1 = pl.pallas_call(
        _base1_body,
        grid=(N // BN,),
        in_specs=[
            pl.BlockSpec((BN, D_IN), lambda i: (i, 0)),
            pl.BlockSpec((D_IN, HID), lambda i: (0, 0)),
        ],
        out_specs=pl.BlockSpec((BN, HID), lambda i: (i, 0)),
        out_shape=jax.ShapeDtypeStruct((N, HID), jnp.float32),
    )(emb16, root1.astype(jnp.bfloat16))

    w_e, acc1p = _edge1_kernel(gidx2d, dst2d, cidx2d, inv,
                               h1b.reshape(R * N, HID // 2))

    h2b = pl.pallas_call(
        _l2_body,
        grid=(N // BN,),
        in_specs=[
            pl.BlockSpec((BN, HID), lambda i: (i, 0)),
            pl.BlockSpec((2, BN, HID), lambda i: (0, i, 0)),
            pl.BlockSpec((8, HID), lambda i: (0, 0)),
            pl.BlockSpec((R + 1, HID, LBL), lambda i: (0, 0, 0)),
        ],
        out_specs=pl.BlockSpec((R + 1, BN, LBL), lambda i: (0, i, 0)),
        out_shape=jax.ShapeDtypeStruct((R + 1, N, LBL), jnp.float32),
    )(base1, acc1p, bias1b, W2cat)

    acc2p = _edge2_kernel(gidx2d, dst2d, w_e,
                          h2b.reshape((R + 1) * N, LBL))

    out = pl.pallas_call(
        _out_body,
        grid=(N // BN,),
        in_specs=[
            pl.BlockSpec((1, BN, LBL), lambda i: (R, i, 0)),
            pl.BlockSpec((2, BN, LBL), lambda i: (0, i, 0)),
            pl.BlockSpec((8, LBL), lambda i: (0, 0)),
        ],
        out_specs=pl.BlockSpec((BN, LBL), lambda i: (i, 0)),
        out_shape=jax.ShapeDtypeStruct((N, LBL), jnp.float32),
    )(h2b, acc2p, bias2b)

    return out
